# baseline (speedup 1.0000x reference)
import functools

import jax
import jax.numpy as jnp
from jax import lax
from jax.experimental import pallas as pl
from jax.experimental.pallas import tpu as pltpu
from jax.experimental.pallas import tpu_sc as plsc

N = 4096
D = 512
L = 8
NSEG = 2 * L
E = 4096
BLK = 64
MMR = 256
E_PAD = 2 * E + NSEG * BLK
NBLK = E_PAD // BLK
NC, NS = 2, 16
NW = NC * NS
GPT = E_PAD // NW
GCH = 72
NGCH = GPT // GCH
TRASH = N
OUT_ROWS = N + 8
SCH = 512
NSC = E_PAD // SCH

_f32 = jnp.float32
_i32 = jnp.int32


def _plan_edges(triples):
    dep = triples[:, 0]
    lab = jnp.remainder(triples[:, 1], L)
    gov = triples[:, 2]
    counts = jnp.zeros((L,), _i32).at[lab].add(1)
    seg_counts = jnp.concatenate([counts, counts])
    padded = ((seg_counts + BLK - 1) // BLK) * BLK
    ends = jnp.cumsum(padded)
    pad_start = ends - padded
    order = jnp.argsort(lab, stable=True)
    lab_s = lab[order]
    seg_start = jnp.cumsum(counts) - counts
    rank = jnp.arange(E, dtype=_i32) - seg_start[lab_s]
    slot_f = pad_start[lab_s] + rank
    slot_r = pad_start[L + lab_s] + rank
    src = jnp.zeros((E_PAD,), _i32)
    dst = jnp.full((E_PAD,), TRASH, _i32)
    src = src.at[slot_f].set(gov[order]).at[slot_r].set(dep[order])
    dst = dst.at[slot_f].set(dep[order]).at[slot_r].set(gov[order])
    block_start = jnp.arange(NBLK, dtype=_i32) * BLK
    wid = jnp.searchsorted(ends, block_start, side="right").astype(_i32)
    wid = jnp.minimum(wid, NSEG - 1)
    return src, dst, wid


def _dense_mm_body(relu, x_ref, w_ref, b_ref, y_ref):
    x = x_ref[...]
    if relu:
        x = jnp.maximum(x, 0.0)
    y_ref[...] = lax.dot_general(x, w_ref[...], (((1,), (1,)), ((), ())),
                                 preferred_element_type=_f32) + b_ref[...]


def _dense_mm(x, w, b, relu):
    return pl.pallas_call(
        functools.partial(_dense_mm_body, relu),
        grid=(N // MMR,),
        in_specs=[
            pl.BlockSpec((MMR, D), lambda i: (i, 0)),
            pl.BlockSpec((D, D), lambda i: (0, 0)),
            pl.BlockSpec((1, D), lambda i: (0, 0)),
        ],
        out_specs=pl.BlockSpec((MMR, D), lambda i: (i, 0)),
        out_shape=jax.ShapeDtypeStruct((N, D), _f32),
    )(x, w, b.reshape(1, D))


def _grouped_mm_body(relu, wid_ref, g_ref, w_ref, b_ref, m_ref):
    del wid_ref
    x = g_ref[...]
    if relu:
        x = jnp.maximum(x, 0.0)
    m_ref[...] = lax.dot_general(x, w_ref[0], (((1,), (1,)), ((), ())),
                                 preferred_element_type=_f32) + b_ref[0]


def _grouped_mm(g, w_dep, b_dep, wid, relu):
    grid_spec = pltpu.PrefetchScalarGridSpec(
        num_scalar_prefetch=1,
        grid=(NBLK,),
        in_specs=[
            pl.BlockSpec((BLK, D), lambda i, wid: (i, 0)),
            pl.BlockSpec((1, D, D), lambda i, wid: (wid[i], 0, 0)),
            pl.BlockSpec((1, 1, D), lambda i, wid: (wid[i], 0, 0)),
        ],
        out_specs=pl.BlockSpec((BLK, D), lambda i, wid: (i, 0)),
    )
    return pl.pallas_call(
        functools.partial(_grouped_mm_body, relu),
        grid_spec=grid_spec,
        out_shape=jax.ShapeDtypeStruct((E_PAD, D), _f32),
    )(wid, g, w_dep, b_dep.reshape(NSEG, 1, D))


def _sc_gather_kernel(x_hbm, idx_hbm, g_hbm, buf0, buf1, buf2, idx_v,
                      sems_g, sems_o):
    c = lax.axis_index("c")
    s = lax.axis_index("s")
    w = c * NS + s
    bufs = (buf0, buf1, buf2)
    base0 = w * GPT
    pltpu.sync_copy(idx_hbm.at[pl.ds(base0, GPT)], idx_v)
    g_descs = [None] * NGCH
    o_descs = [None] * NGCH

    def start_o(k):
        g_descs[k].wait()
        o_descs[k] = pltpu.async_copy(
            bufs[k % 3], g_hbm.at[pl.ds(base0 + k * GCH, GCH)],
            sems_o.at[k % 3])

    for k in range(NGCH):
        r = k % 3
        if k >= 3:
            o_descs[k - 3].wait()
        g_descs[k] = pltpu.async_copy(
            x_hbm.at[idx_v.at[pl.ds(k * GCH, GCH)]], bufs[r], sems_g.at[r])
        if k >= 2:
            start_o(k - 2)
    for k in range(max(0, NGCH - 2), NGCH):
        start_o(k)
    for k in range(max(0, NGCH - 3), NGCH):
        o_descs[k].wait()


def _sc_gather(x, src):
    dt = x.dtype
    mesh = plsc.VectorSubcoreMesh(core_axis_name="c", subcore_axis_name="s")
    kern = pl.kernel(
        _sc_gather_kernel,
        out_type=jax.ShapeDtypeStruct((E_PAD, D), dt),
        mesh=mesh,
        scratch_types=[
            pltpu.VMEM((GCH, D), dt),
            pltpu.VMEM((GCH, D), dt),
            pltpu.VMEM((GCH, D), dt),
            pltpu.VMEM((GPT,), _i32),
            pltpu.SemaphoreType.DMA((3,)),
            pltpu.SemaphoreType.DMA((3,)),
        ],
    )
    return kern(x, src)


def _sc_scatter_gather_kernel(m_hbm, t_hbm, idx_hbm, src_hbm,
                              o_hbm, g2_hbm,
                              acc_v, bufa, bufb, idx_v, src_v,
                              gbufa, gbufb, sem_a, sem_b, sem_g):
    c = lax.axis_index("c")
    s = lax.axis_index("s")
    col = (c * NS + s) * 16
    iota16 = lax.iota(_i32, 16)

    pltpu.sync_copy(idx_hbm, idx_v)
    pltpu.sync_copy(src_hbm, src_v)
    pltpu.sync_copy(t_hbm.at[:, pl.ds(col, 16)], acc_v.at[pl.ds(0, N)])
    pltpu.async_copy(m_hbm.at[pl.ds(0, SCH), pl.ds(col, 16)], bufa, sem_a)

    def process(buf, base):
        @plsc.parallel_loop(0, SCH // 16, unroll=2)
        def group(g):
            vrow = idx_v[pl.ds(base + g * 16, 16)]
            for j in range(16):
                rowb = vrow[jnp.full((16,), j, _i32)]
                vals = buf[g * 16 + j, :]
                plsc.addupdate_scatter(acc_v, [rowb, iota16], vals)

    def pair(i, _):
        base_a = (2 * i) * SCH
        pltpu.make_async_copy(m_hbm.at[pl.ds(base_a, SCH), pl.ds(col, 16)],
                              bufa, sem_a).wait()
        pltpu.async_copy(m_hbm.at[pl.ds(base_a + SCH, SCH), pl.ds(col, 16)],
                         bufb, sem_b)
        process(bufa, base_a)
        pltpu.make_async_copy(m_hbm.at[pl.ds(base_a + SCH, SCH), pl.ds(col, 16)],
                              bufb, sem_b).wait()

        @pl.when(i < NSC // 2 - 1)
        def _():
            pltpu.async_copy(
                m_hbm.at[pl.ds(base_a + 2 * SCH, SCH), pl.ds(col, 16)],
                bufa, sem_a)

        process(bufb, base_a + SCH)
        return 0

    lax.fori_loop(0, NSC // 2, pair, 0)
    pltpu.sync_copy(acc_v.at[pl.ds(0, N)], o_hbm.at[:, pl.ds(col, 16)])

    def fill(gbuf, base):
        @plsc.parallel_loop(0, SCH // 16, unroll=2)
        def group(g):
            vsrc = src_v[pl.ds(base + g * 16, 16)]
            for j in range(16):
                rowb = vsrc[jnp.full((16,), j, _i32)]
                gbuf[g * 16 + j, :] = plsc.load_gather(acc_v, [rowb, iota16])

    def gpair(i, _):
        base_a = (2 * i) * SCH

        @pl.when(i > 0)
        def _():
            pltpu.make_async_copy(
                gbufa, g2_hbm.at[pl.ds(base_a - 2 * SCH, SCH), pl.ds(col, 16)],
                sem_g).wait()

        fill(gbufa, base_a)
        pltpu.async_copy(gbufa, g2_hbm.at[pl.ds(base_a, SCH), pl.ds(col, 16)],
                         sem_g)

        @pl.when(i > 0)
        def _():
            pltpu.make_async_copy(
                gbufb, g2_hbm.at[pl.ds(base_a - SCH, SCH), pl.ds(col, 16)],
                sem_g).wait()

        fill(gbufb, base_a + SCH)
        pltpu.async_copy(gbufb,
                         g2_hbm.at[pl.ds(base_a + SCH, SCH), pl.ds(col, 16)],
                         sem_g)
        return 0

    lax.fori_loop(0, NSC // 2, gpair, 0)
    pltpu.make_async_copy(
        gbufa, g2_hbm.at[pl.ds(E_PAD - 2 * SCH, SCH), pl.ds(col, 16)],
        sem_g).wait()
    pltpu.make_async_copy(
        gbufb, g2_hbm.at[pl.ds(E_PAD - SCH, SCH), pl.ds(col, 16)],
        sem_g).wait()


def _sc_scatter_gather(m, t, dst, src):
    mesh = plsc.VectorSubcoreMesh(core_axis_name="c", subcore_axis_name="s")
    kern = pl.kernel(
        _sc_scatter_gather_kernel,
        out_type=(
            jax.ShapeDtypeStruct((N, D), _f32),
            jax.ShapeDtypeStruct((E_PAD, D), _f32),
        ),
        mesh=mesh,
        scratch_types=[
            pltpu.VMEM((OUT_ROWS, 16), _f32),
            pltpu.VMEM((SCH, 16), _f32),
            pltpu.VMEM((SCH, 16), _f32),
            pltpu.VMEM((E_PAD,), _i32),
            pltpu.VMEM((E_PAD,), _i32),
            pltpu.VMEM((SCH, 16), _f32),
            pltpu.VMEM((SCH, 16), _f32),
            pltpu.SemaphoreType.DMA,
            pltpu.SemaphoreType.DMA,
            pltpu.SemaphoreType.DMA,
        ],
        compiler_params=pltpu.CompilerParams(
            needs_layout_passes=False, use_tc_tiling_on_sc=False),
    )
    return kern(m, t, dst, src)


def _sc_scatter_kernel(m_hbm, t_hbm, idx_hbm, o_hbm,
                       acc_v, bufa, bufb, idx_v, sem_a, sem_b):
    c = lax.axis_index("c")
    s = lax.axis_index("s")
    col = (c * NS + s) * 16
    iota16 = lax.iota(_i32, 16)

    pltpu.sync_copy(idx_hbm, idx_v)
    pltpu.sync_copy(t_hbm.at[:, pl.ds(col, 16)], acc_v.at[pl.ds(0, N)])
    pltpu.async_copy(m_hbm.at[pl.ds(0, SCH), pl.ds(col, 16)], bufa, sem_a)

    def process(buf, base):
        @plsc.parallel_loop(0, SCH // 16, unroll=2)
        def group(g):
            vrow = idx_v[pl.ds(base + g * 16, 16)]
            for j in range(16):
                rowb = vrow[jnp.full((16,), j, _i32)]
                vals = buf[g * 16 + j, :]
                plsc.addupdate_scatter(acc_v, [rowb, iota16], vals)

    def pair(i, _):
        base_a = (2 * i) * SCH
        pltpu.make_async_copy(m_hbm.at[pl.ds(base_a, SCH), pl.ds(col, 16)],
                              bufa, sem_a).wait()
        pltpu.async_copy(m_hbm.at[pl.ds(base_a + SCH, SCH), pl.ds(col, 16)],
                         bufb, sem_b)
        process(bufa, base_a)
        pltpu.make_async_copy(m_hbm.at[pl.ds(base_a + SCH, SCH), pl.ds(col, 16)],
                              bufb, sem_b).wait()

        @pl.when(i < NSC // 2 - 1)
        def _():
            pltpu.async_copy(
                m_hbm.at[pl.ds(base_a + 2 * SCH, SCH), pl.ds(col, 16)],
                bufa, sem_a)

        process(bufb, base_a + SCH)
        return 0

    lax.fori_loop(0, NSC // 2, pair, 0)
    pltpu.sync_copy(acc_v.at[pl.ds(0, N)], o_hbm.at[:, pl.ds(col, 16)])


def _sc_scatter(m, t, dst):
    mesh = plsc.VectorSubcoreMesh(core_axis_name="c", subcore_axis_name="s")
    kern = pl.kernel(
        _sc_scatter_kernel,
        out_type=jax.ShapeDtypeStruct((N, D), _f32),
        mesh=mesh,
        scratch_types=[
            pltpu.VMEM((OUT_ROWS, 16), _f32),
            pltpu.VMEM((SCH, 16), _f32),
            pltpu.VMEM((SCH, 16), _f32),
            pltpu.VMEM((E_PAD,), _i32),
            pltpu.SemaphoreType.DMA,
            pltpu.SemaphoreType.DMA,
        ],
        compiler_params=pltpu.CompilerParams(
            needs_layout_passes=False, use_tc_tiling_on_sc=False),
    )
    return kern(m, t, dst)


def kernel(_input, dependency_triples, W_self, b_self, W_dep, b_dep, W_ff, b_ff):
    src, dst, wid = _plan_edges(dependency_triples)
    t1 = _dense_mm(_input, W_self[0], b_self[0], False)
    g1 = _sc_gather(_input, src)
    m1 = _grouped_mm(g1, W_dep[0], b_dep[0], wid, False)
    x1, g2 = _sc_scatter_gather(m1, t1, dst, src)
    t2 = _dense_mm(x1, W_self[1], b_self[1], True)
    m2 = _grouped_mm(g2, W_dep[1], b_dep[1], wid, True)
    x2 = _sc_scatter(m2, t2, dst)
    return _dense_mm(x2, W_ff, b_ff, True)


# speedup vs baseline: 1.0654x; 1.0654x over previous
import functools

import jax
import jax.numpy as jnp
from jax import lax
from jax.experimental import pallas as pl
from jax.experimental.pallas import tpu as pltpu
from jax.experimental.pallas import tpu_sc as plsc

N = 4096
D = 512
L = 8
NSEG = 2 * L
E = 4096
BLK = 128
MMR = 256
E_PAD = 2 * E + NSEG * BLK
NBLK = E_PAD // BLK
NC, NS = 2, 16
NW = NC * NS
GPT = E_PAD // NW
GCH = 80
NGCH = GPT // GCH
TRASH = N
OUT_ROWS = N + 8
SCH = 512
NSC = E_PAD // SCH

_f32 = jnp.float32
_i32 = jnp.int32


def _plan_edges(triples):
    dep = triples[:, 0]
    lab = jnp.remainder(triples[:, 1], L)
    gov = triples[:, 2]
    counts = jnp.zeros((L,), _i32).at[lab].add(1)
    seg_counts = jnp.concatenate([counts, counts])
    padded = ((seg_counts + BLK - 1) // BLK) * BLK
    ends = jnp.cumsum(padded)
    pad_start = ends - padded
    order = jnp.argsort(lab, stable=True)
    lab_s = lab[order]
    seg_start = jnp.cumsum(counts) - counts
    rank = jnp.arange(E, dtype=_i32) - seg_start[lab_s]
    slot_f = pad_start[lab_s] + rank
    slot_r = pad_start[L + lab_s] + rank
    src = jnp.zeros((E_PAD,), _i32)
    dst = jnp.full((E_PAD,), TRASH, _i32)
    src = src.at[slot_f].set(gov[order]).at[slot_r].set(dep[order])
    dst = dst.at[slot_f].set(dep[order]).at[slot_r].set(gov[order])
    block_start = jnp.arange(NBLK, dtype=_i32) * BLK
    wid = jnp.searchsorted(ends, block_start, side="right").astype(_i32)
    wid = jnp.minimum(wid, NSEG - 1)
    return src, dst, wid


def _dense_mm_body(relu, x_ref, w_ref, b_ref, y_ref):
    x = x_ref[...]
    if relu:
        x = jnp.maximum(x, 0.0)
    y_ref[...] = lax.dot_general(x, w_ref[...], (((1,), (1,)), ((), ())),
                                 preferred_element_type=_f32) + b_ref[...]


def _dense_mm(x, w, b, relu):
    return pl.pallas_call(
        functools.partial(_dense_mm_body, relu),
        grid=(N // MMR,),
        in_specs=[
            pl.BlockSpec((MMR, D), lambda i: (i, 0)),
            pl.BlockSpec((D, D), lambda i: (0, 0)),
            pl.BlockSpec((1, D), lambda i: (0, 0)),
        ],
        out_specs=pl.BlockSpec((MMR, D), lambda i: (i, 0)),
        out_shape=jax.ShapeDtypeStruct((N, D), _f32),
    )(x, w, b.reshape(1, D))


def _grouped_mm_body(relu, wid_ref, g_ref, w_ref, b_ref, m_ref):
    del wid_ref
    x = g_ref[...]
    if relu:
        x = jnp.maximum(x, 0.0)
    m_ref[...] = lax.dot_general(x, w_ref[0], (((1,), (1,)), ((), ())),
                                 preferred_element_type=_f32) + b_ref[0]


def _grouped_mm(g, w_dep, b_dep, wid, relu):
    rows = g.shape[0]
    grid_spec = pltpu.PrefetchScalarGridSpec(
        num_scalar_prefetch=1,
        grid=(rows // BLK,),
        in_specs=[
            pl.BlockSpec((BLK, D), lambda i, wid: (i, 0)),
            pl.BlockSpec((1, D, D), lambda i, wid: (wid[i], 0, 0)),
            pl.BlockSpec((1, 1, D), lambda i, wid: (wid[i], 0, 0)),
        ],
        out_specs=pl.BlockSpec((BLK, D), lambda i, wid: (i, 0)),
    )
    return pl.pallas_call(
        functools.partial(_grouped_mm_body, relu),
        grid_spec=grid_spec,
        out_shape=jax.ShapeDtypeStruct((rows, D), _f32),
    )(wid, g, w_dep, b_dep.reshape(NSEG, 1, D))


def _sc_gather_kernel(gpt, ngch, x_hbm, idx_hbm, g_hbm, buf0, buf1, buf2,
                      idx_v, sems_g, sems_o):
    c = lax.axis_index("c")
    s = lax.axis_index("s")
    w = c * NS + s
    bufs = (buf0, buf1, buf2)
    base0 = w * gpt
    pltpu.sync_copy(idx_hbm.at[pl.ds(base0, gpt)], idx_v)
    g_descs = [None] * ngch
    o_descs = [None] * ngch

    def start_o(k):
        g_descs[k].wait()
        o_descs[k] = pltpu.async_copy(
            bufs[k % 3], g_hbm.at[pl.ds(base0 + k * GCH, GCH)],
            sems_o.at[k % 3])

    for k in range(ngch):
        r = k % 3
        if k >= 3:
            o_descs[k - 3].wait()
        g_descs[k] = pltpu.async_copy(
            x_hbm.at[idx_v.at[pl.ds(k * GCH, GCH)]], bufs[r], sems_g.at[r])
        if k >= 2:
            start_o(k - 2)
    for k in range(max(0, ngch - 2), ngch):
        start_o(k)
    for k in range(max(0, ngch - 3), ngch):
        o_descs[k].wait()


def _sc_gather(x, src):
    dt = x.dtype
    rows = src.shape[0]
    gpt = rows // NW
    ngch = gpt // GCH
    mesh = plsc.VectorSubcoreMesh(core_axis_name="c", subcore_axis_name="s")
    kern = pl.kernel(
        functools.partial(_sc_gather_kernel, gpt, ngch),
        out_type=jax.ShapeDtypeStruct((rows, D), dt),
        mesh=mesh,
        scratch_types=[
            pltpu.VMEM((GCH, D), dt),
            pltpu.VMEM((GCH, D), dt),
            pltpu.VMEM((GCH, D), dt),
            pltpu.VMEM((gpt,), _i32),
            pltpu.SemaphoreType.DMA((3,)),
            pltpu.SemaphoreType.DMA((3,)),
        ],
    )
    return kern(x, src)


def _sc_scatter_gather_kernel(ma_hbm, mb_hbm, t_hbm, idx_hbm, src_hbm,
                              o_hbm, g2_hbm,
                              acc_v, bufa, bufb, idx_v, src_v,
                              gbufa, gbufb, sem_a, sem_b, sem_g):
    c = lax.axis_index("c")
    s = lax.axis_index("s")
    col = (c * NS + s) * 16
    iota16 = lax.iota(_i32, 16)
    halfn = E_PAD // 2
    hpairs = halfn // SCH // 2

    pltpu.sync_copy(idx_hbm, idx_v)
    pltpu.sync_copy(src_hbm, src_v)
    pltpu.sync_copy(t_hbm.at[:, pl.ds(col, 16)], acc_v.at[pl.ds(0, N)])

    def process(buf, base):
        @plsc.parallel_loop(0, SCH // 16, unroll=2)
        def group(g):
            vrow = idx_v[pl.ds(base + g * 16, 16)]
            for j in range(16):
                rowb = vrow[jnp.full((16,), j, _i32)]
                vals = buf[g * 16 + j, :]
                plsc.addupdate_scatter(acc_v, [rowb, iota16], vals)

    def run_half(m_hbm, goff):
        pltpu.async_copy(m_hbm.at[pl.ds(0, SCH), pl.ds(col, 16)], bufa, sem_a)

        def pair(i, _):
            base_a = (2 * i) * SCH
            pltpu.make_async_copy(m_hbm.at[pl.ds(base_a, SCH), pl.ds(col, 16)],
                                  bufa, sem_a).wait()
            pltpu.async_copy(
                m_hbm.at[pl.ds(base_a + SCH, SCH), pl.ds(col, 16)],
                bufb, sem_b)
            process(bufa, goff + base_a)
            pltpu.make_async_copy(
                m_hbm.at[pl.ds(base_a + SCH, SCH), pl.ds(col, 16)],
                bufb, sem_b).wait()

            @pl.when(i < hpairs - 1)
            def _():
                pltpu.async_copy(
                    m_hbm.at[pl.ds(base_a + 2 * SCH, SCH), pl.ds(col, 16)],
                    bufa, sem_a)

            process(bufb, goff + base_a + SCH)
            return 0

        lax.fori_loop(0, hpairs, pair, 0)

    run_half(ma_hbm, 0)
    run_half(mb_hbm, halfn)
    pltpu.sync_copy(acc_v.at[pl.ds(0, N)], o_hbm.at[:, pl.ds(col, 16)])

    def fill(gbuf, base):
        @plsc.parallel_loop(0, SCH // 16, unroll=2)
        def group(g):
            vsrc = src_v[pl.ds(base + g * 16, 16)]
            for j in range(16):
                rowb = vsrc[jnp.full((16,), j, _i32)]
                gbuf[g * 16 + j, :] = plsc.load_gather(acc_v, [rowb, iota16])

    def gpair(i, _):
        base_a = (2 * i) * SCH

        @pl.when(i > 0)
        def _():
            pltpu.make_async_copy(
                gbufa, g2_hbm.at[pl.ds(base_a - 2 * SCH, SCH), pl.ds(col, 16)],
                sem_g).wait()

        fill(gbufa, base_a)
        pltpu.async_copy(gbufa, g2_hbm.at[pl.ds(base_a, SCH), pl.ds(col, 16)],
                         sem_g)

        @pl.when(i > 0)
        def _():
            pltpu.make_async_copy(
                gbufb, g2_hbm.at[pl.ds(base_a - SCH, SCH), pl.ds(col, 16)],
                sem_g).wait()

        fill(gbufb, base_a + SCH)
        pltpu.async_copy(gbufb,
                         g2_hbm.at[pl.ds(base_a + SCH, SCH), pl.ds(col, 16)],
                         sem_g)
        return 0

    lax.fori_loop(0, NSC // 2, gpair, 0)
    pltpu.make_async_copy(
        gbufa, g2_hbm.at[pl.ds(E_PAD - 2 * SCH, SCH), pl.ds(col, 16)],
        sem_g).wait()
    pltpu.make_async_copy(
        gbufb, g2_hbm.at[pl.ds(E_PAD - SCH, SCH), pl.ds(col, 16)],
        sem_g).wait()


def _sc_scatter_gather(ma, mb, t, dst, src):
    mesh = plsc.VectorSubcoreMesh(core_axis_name="c", subcore_axis_name="s")
    kern = pl.kernel(
        _sc_scatter_gather_kernel,
        out_type=(
            jax.ShapeDtypeStruct((N, D), _f32),
            jax.ShapeDtypeStruct((E_PAD, D), _f32),
        ),
        mesh=mesh,
        scratch_types=[
            pltpu.VMEM((OUT_ROWS, 16), _f32),
            pltpu.VMEM((SCH, 16), _f32),
            pltpu.VMEM((SCH, 16), _f32),
            pltpu.VMEM((E_PAD,), _i32),
            pltpu.VMEM((E_PAD,), _i32),
            pltpu.VMEM((SCH, 16), _f32),
            pltpu.VMEM((SCH, 16), _f32),
            pltpu.SemaphoreType.DMA,
            pltpu.SemaphoreType.DMA,
            pltpu.SemaphoreType.DMA,
        ],
        compiler_params=pltpu.CompilerParams(
            needs_layout_passes=False, use_tc_tiling_on_sc=False),
    )
    return kern(ma, mb, t, dst, src)


def _sc_scatter_kernel(m_hbm, t_hbm, idx_hbm, o_hbm,
                       acc_v, bufa, bufb, idx_v, sem_a, sem_b):
    c = lax.axis_index("c")
    s = lax.axis_index("s")
    col = (c * NS + s) * 16
    iota16 = lax.iota(_i32, 16)

    pltpu.sync_copy(idx_hbm, idx_v)
    pltpu.sync_copy(t_hbm.at[:, pl.ds(col, 16)], acc_v.at[pl.ds(0, N)])
    pltpu.async_copy(m_hbm.at[pl.ds(0, SCH), pl.ds(col, 16)], bufa, sem_a)

    def process(buf, base):
        @plsc.parallel_loop(0, SCH // 16, unroll=2)
        def group(g):
            vrow = idx_v[pl.ds(base + g * 16, 16)]
            for j in range(16):
                rowb = vrow[jnp.full((16,), j, _i32)]
                vals = buf[g * 16 + j, :]
                plsc.addupdate_scatter(acc_v, [rowb, iota16], vals)

    def pair(i, _):
        base_a = (2 * i) * SCH
        pltpu.make_async_copy(m_hbm.at[pl.ds(base_a, SCH), pl.ds(col, 16)],
                              bufa, sem_a).wait()
        pltpu.async_copy(m_hbm.at[pl.ds(base_a + SCH, SCH), pl.ds(col, 16)],
                         bufb, sem_b)
        process(bufa, base_a)
        pltpu.make_async_copy(m_hbm.at[pl.ds(base_a + SCH, SCH), pl.ds(col, 16)],
                              bufb, sem_b).wait()

        @pl.when(i < NSC // 2 - 1)
        def _():
            pltpu.async_copy(
                m_hbm.at[pl.ds(base_a + 2 * SCH, SCH), pl.ds(col, 16)],
                bufa, sem_a)

        process(bufb, base_a + SCH)
        return 0

    lax.fori_loop(0, NSC // 2, pair, 0)
    pltpu.sync_copy(acc_v.at[pl.ds(0, N)], o_hbm.at[:, pl.ds(col, 16)])


def _sc_scatter(m, t, dst):
    mesh = plsc.VectorSubcoreMesh(core_axis_name="c", subcore_axis_name="s")
    kern = pl.kernel(
        _sc_scatter_kernel,
        out_type=jax.ShapeDtypeStruct((N, D), _f32),
        mesh=mesh,
        scratch_types=[
            pltpu.VMEM((OUT_ROWS, 16), _f32),
            pltpu.VMEM((SCH, 16), _f32),
            pltpu.VMEM((SCH, 16), _f32),
            pltpu.VMEM((E_PAD,), _i32),
            pltpu.SemaphoreType.DMA,
            pltpu.SemaphoreType.DMA,
        ],
        compiler_params=pltpu.CompilerParams(
            needs_layout_passes=False, use_tc_tiling_on_sc=False),
    )
    return kern(m, t, dst)


def kernel(_input, dependency_triples, W_self, b_self, W_dep, b_dep, W_ff, b_ff):
    src, dst, wid = _plan_edges(dependency_triples)
    half = E_PAD // 2
    t1 = _dense_mm(_input, W_self[0], b_self[0], False)
    g1a = _sc_gather(_input, src[:half])
    m1a = _grouped_mm(g1a, W_dep[0], b_dep[0], wid[:NBLK // 2], False)
    g1b = _sc_gather(_input, src[half:])
    m1b = _grouped_mm(g1b, W_dep[0], b_dep[0], wid[NBLK // 2:], False)
    x1, g2 = _sc_scatter_gather(m1a, m1b, t1, dst, src)
    t2 = _dense_mm(x1, W_self[1], b_self[1], True)
    m2 = _grouped_mm(g2, W_dep[1], b_dep[1], wid, True)
    x2 = _sc_scatter(m2, t2, dst)
    return _dense_mm(x2, W_ff, b_ff, True)


# speedup vs baseline: 1.0712x; 1.0055x over previous
import functools

import jax
import jax.numpy as jnp
from jax import lax
from jax.experimental import pallas as pl
from jax.experimental.pallas import tpu as pltpu
from jax.experimental.pallas import tpu_sc as plsc

N = 4096
D = 512
L = 8
NSEG = 2 * L
E = 4096
BLK = 128
MMR = 256
E_PAD = 2 * E + NSEG * BLK
NBLK = E_PAD // BLK
NC, NS = 2, 16
NW = NC * NS
GPT = E_PAD // NW
GCH = 80
NGCH = GPT // GCH
TRASH = N
OUT_ROWS = N + 8
SCH = 512
NSC = E_PAD // SCH

_f32 = jnp.float32
_i32 = jnp.int32


def _plan_edges(triples):
    dep = triples[:, 0]
    lab = jnp.remainder(triples[:, 1], L)
    gov = triples[:, 2]
    counts = jnp.zeros((L,), _i32).at[lab].add(1)
    seg_counts = jnp.concatenate([counts, counts])
    padded = ((seg_counts + BLK - 1) // BLK) * BLK
    ends = jnp.cumsum(padded)
    pad_start = ends - padded
    order = jnp.argsort(lab, stable=True)
    lab_s = lab[order]
    seg_start = jnp.cumsum(counts) - counts
    rank = jnp.arange(E, dtype=_i32) - seg_start[lab_s]
    slot_f = pad_start[lab_s] + rank
    slot_r = pad_start[L + lab_s] + rank
    src = jnp.zeros((E_PAD,), _i32)
    dst = jnp.full((E_PAD,), TRASH, _i32)
    src = src.at[slot_f].set(gov[order]).at[slot_r].set(dep[order])
    dst = dst.at[slot_f].set(dep[order]).at[slot_r].set(gov[order])
    block_start = jnp.arange(NBLK, dtype=_i32) * BLK
    wid = jnp.searchsorted(ends, block_start, side="right").astype(_i32)
    wid = jnp.minimum(wid, NSEG - 1)
    return src, dst, wid


def _dense_mm_body(relu, x_ref, w_ref, b_ref, y_ref):
    x = x_ref[...]
    if relu:
        x = jnp.maximum(x, 0.0)
    y_ref[...] = lax.dot_general(x, w_ref[...], (((1,), (1,)), ((), ())),
                                 preferred_element_type=_f32) + b_ref[...]


def _dense_mm(x, w, b, relu):
    return pl.pallas_call(
        functools.partial(_dense_mm_body, relu),
        grid=(N // MMR,),
        in_specs=[
            pl.BlockSpec((MMR, D), lambda i: (i, 0)),
            pl.BlockSpec((D, D), lambda i: (0, 0)),
            pl.BlockSpec((1, D), lambda i: (0, 0)),
        ],
        out_specs=pl.BlockSpec((MMR, D), lambda i: (i, 0)),
        out_shape=jax.ShapeDtypeStruct((N, D), _f32),
    )(x, w, b.reshape(1, D))


def _grouped_mm_body(relu, wid_ref, g_ref, w_ref, b_ref, m_ref):
    del wid_ref
    x = g_ref[...]
    if relu:
        x = jnp.maximum(x, 0.0)
    m_ref[...] = lax.dot_general(x, w_ref[0], (((1,), (1,)), ((), ())),
                                 preferred_element_type=_f32) + b_ref[0]


def _grouped_mm(g, w_dep, b_dep, wid, relu):
    rows = g.shape[0]
    grid_spec = pltpu.PrefetchScalarGridSpec(
        num_scalar_prefetch=1,
        grid=(rows // BLK,),
        in_specs=[
            pl.BlockSpec((BLK, D), lambda i, wid: (i, 0)),
            pl.BlockSpec((1, D, D), lambda i, wid: (wid[i], 0, 0)),
            pl.BlockSpec((1, 1, D), lambda i, wid: (wid[i], 0, 0)),
        ],
        out_specs=pl.BlockSpec((BLK, D), lambda i, wid: (i, 0)),
    )
    return pl.pallas_call(
        functools.partial(_grouped_mm_body, relu),
        grid_spec=grid_spec,
        out_shape=jax.ShapeDtypeStruct((rows, D), _f32),
    )(wid, g, w_dep, b_dep.reshape(NSEG, 1, D))


def _sc_gather_kernel(gpt, ngch, x_hbm, idx_hbm, g_hbm, buf0, buf1, buf2,
                      idx_v, sems_g, sems_o):
    c = lax.axis_index("c")
    s = lax.axis_index("s")
    w = c * NS + s
    bufs = (buf0, buf1, buf2)
    base0 = w * gpt
    pltpu.sync_copy(idx_hbm.at[pl.ds(base0, gpt)], idx_v)
    g_descs = [None] * ngch
    o_descs = [None] * ngch

    def start_o(k):
        g_descs[k].wait()
        o_descs[k] = pltpu.async_copy(
            bufs[k % 3], g_hbm.at[pl.ds(base0 + k * GCH, GCH)],
            sems_o.at[k % 3])

    for k in range(ngch):
        r = k % 3
        if k >= 3:
            o_descs[k - 3].wait()
        g_descs[k] = pltpu.async_copy(
            x_hbm.at[idx_v.at[pl.ds(k * GCH, GCH)]], bufs[r], sems_g.at[r])
        if k >= 2:
            start_o(k - 2)
    for k in range(max(0, ngch - 2), ngch):
        start_o(k)
    for k in range(max(0, ngch - 3), ngch):
        o_descs[k].wait()


def _sc_gather(x, src):
    dt = x.dtype
    rows = src.shape[0]
    gpt = rows // NW
    ngch = gpt // GCH
    mesh = plsc.VectorSubcoreMesh(core_axis_name="c", subcore_axis_name="s")
    kern = pl.kernel(
        functools.partial(_sc_gather_kernel, gpt, ngch),
        out_type=jax.ShapeDtypeStruct((rows, D), dt),
        mesh=mesh,
        scratch_types=[
            pltpu.VMEM((GCH, D), dt),
            pltpu.VMEM((GCH, D), dt),
            pltpu.VMEM((GCH, D), dt),
            pltpu.VMEM((gpt,), _i32),
            pltpu.SemaphoreType.DMA((3,)),
            pltpu.SemaphoreType.DMA((3,)),
        ],
    )
    return kern(x, src)


def _sc_scatter_gather_kernel(m_hbm, t_hbm, idx_hbm, src_hbm,
                              o_hbm, g2_hbm,
                              acc_v, bufa, bufb, idx_v, src_v,
                              gbufa, gbufb, sem_a, sem_b, sem_g):
    c = lax.axis_index("c")
    s = lax.axis_index("s")
    col = (c * NS + s) * 16
    iota16 = lax.iota(_i32, 16)

    pltpu.sync_copy(idx_hbm, idx_v)
    pltpu.sync_copy(src_hbm, src_v)
    pltpu.sync_copy(t_hbm.at[:, pl.ds(col, 16)], acc_v.at[pl.ds(0, N)])
    pltpu.async_copy(m_hbm.at[pl.ds(0, SCH), pl.ds(col, 16)], bufa, sem_a)

    def process(buf, base):
        @plsc.parallel_loop(0, SCH // 16, unroll=2)
        def group(g):
            vrow = idx_v[pl.ds(base + g * 16, 16)]
            for j in range(16):
                rowb = vrow[jnp.full((16,), j, _i32)]
                vals = buf[g * 16 + j, :]
                plsc.addupdate_scatter(acc_v, [rowb, iota16], vals)

    def pair(i, _):
        base_a = (2 * i) * SCH
        pltpu.make_async_copy(m_hbm.at[pl.ds(base_a, SCH), pl.ds(col, 16)],
                              bufa, sem_a).wait()
        pltpu.async_copy(m_hbm.at[pl.ds(base_a + SCH, SCH), pl.ds(col, 16)],
                         bufb, sem_b)
        process(bufa, base_a)
        pltpu.make_async_copy(m_hbm.at[pl.ds(base_a + SCH, SCH), pl.ds(col, 16)],
                              bufb, sem_b).wait()

        @pl.when(i < NSC // 2 - 1)
        def _():
            pltpu.async_copy(
                m_hbm.at[pl.ds(base_a + 2 * SCH, SCH), pl.ds(col, 16)],
                bufa, sem_a)

        process(bufb, base_a + SCH)
        return 0

    lax.fori_loop(0, NSC // 2, pair, 0)
    pltpu.sync_copy(acc_v.at[pl.ds(0, N)], o_hbm.at[:, pl.ds(col, 16)])

    def fill(gbuf, base):
        @plsc.parallel_loop(0, SCH // 16, unroll=2)
        def group(g):
            vsrc = src_v[pl.ds(base + g * 16, 16)]
            for j in range(16):
                rowb = vsrc[jnp.full((16,), j, _i32)]
                gbuf[g * 16 + j, :] = plsc.load_gather(acc_v, [rowb, iota16])

    def gpair(i, _):
        base_a = (2 * i) * SCH

        @pl.when(i > 0)
        def _():
            pltpu.make_async_copy(
                gbufa, g2_hbm.at[pl.ds(base_a - 2 * SCH, SCH), pl.ds(col, 16)],
                sem_g).wait()

        fill(gbufa, base_a)
        pltpu.async_copy(gbufa, g2_hbm.at[pl.ds(base_a, SCH), pl.ds(col, 16)],
                         sem_g)

        @pl.when(i > 0)
        def _():
            pltpu.make_async_copy(
                gbufb, g2_hbm.at[pl.ds(base_a - SCH, SCH), pl.ds(col, 16)],
                sem_g).wait()

        fill(gbufb, base_a + SCH)
        pltpu.async_copy(gbufb,
                         g2_hbm.at[pl.ds(base_a + SCH, SCH), pl.ds(col, 16)],
                         sem_g)
        return 0

    lax.fori_loop(0, NSC // 2, gpair, 0)
    pltpu.make_async_copy(
        gbufa, g2_hbm.at[pl.ds(E_PAD - 2 * SCH, SCH), pl.ds(col, 16)],
        sem_g).wait()
    pltpu.make_async_copy(
        gbufb, g2_hbm.at[pl.ds(E_PAD - SCH, SCH), pl.ds(col, 16)],
        sem_g).wait()


def _sc_scatter_gather(m, t, dst, src):
    mesh = plsc.VectorSubcoreMesh(core_axis_name="c", subcore_axis_name="s")
    kern = pl.kernel(
        _sc_scatter_gather_kernel,
        out_type=(
            jax.ShapeDtypeStruct((N, D), _f32),
            jax.ShapeDtypeStruct((E_PAD, D), _f32),
        ),
        mesh=mesh,
        scratch_types=[
            pltpu.VMEM((OUT_ROWS, 16), _f32),
            pltpu.VMEM((SCH, 16), _f32),
            pltpu.VMEM((SCH, 16), _f32),
            pltpu.VMEM((E_PAD,), _i32),
            pltpu.VMEM((E_PAD,), _i32),
            pltpu.VMEM((SCH, 16), _f32),
            pltpu.VMEM((SCH, 16), _f32),
            pltpu.SemaphoreType.DMA,
            pltpu.SemaphoreType.DMA,
            pltpu.SemaphoreType.DMA,
        ],
        compiler_params=pltpu.CompilerParams(
            needs_layout_passes=False, use_tc_tiling_on_sc=False),
    )
    return kern(m, t, dst, src)


def _sc_scatter_kernel(m_hbm, t_hbm, idx_hbm, o_hbm,
                       acc_v, bufa, bufb, idx_v, sem_a, sem_b):
    c = lax.axis_index("c")
    s = lax.axis_index("s")
    col = (c * NS + s) * 16
    iota16 = lax.iota(_i32, 16)

    pltpu.sync_copy(idx_hbm, idx_v)
    pltpu.sync_copy(t_hbm.at[:, pl.ds(col, 16)], acc_v.at[pl.ds(0, N)])
    pltpu.async_copy(m_hbm.at[pl.ds(0, SCH), pl.ds(col, 16)], bufa, sem_a)

    def process(buf, base):
        @plsc.parallel_loop(0, SCH // 16, unroll=2)
        def group(g):
            vrow = idx_v[pl.ds(base + g * 16, 16)]
            for j in range(16):
                rowb = vrow[jnp.full((16,), j, _i32)]
                vals = buf[g * 16 + j, :]
                plsc.addupdate_scatter(acc_v, [rowb, iota16], vals)

    def pair(i, _):
        base_a = (2 * i) * SCH
        pltpu.make_async_copy(m_hbm.at[pl.ds(base_a, SCH), pl.ds(col, 16)],
                              bufa, sem_a).wait()
        pltpu.async_copy(m_hbm.at[pl.ds(base_a + SCH, SCH), pl.ds(col, 16)],
                         bufb, sem_b)
        process(bufa, base_a)
        pltpu.make_async_copy(m_hbm.at[pl.ds(base_a + SCH, SCH), pl.ds(col, 16)],
                              bufb, sem_b).wait()

        @pl.when(i < NSC // 2 - 1)
        def _():
            pltpu.async_copy(
                m_hbm.at[pl.ds(base_a + 2 * SCH, SCH), pl.ds(col, 16)],
                bufa, sem_a)

        process(bufb, base_a + SCH)
        return 0

    lax.fori_loop(0, NSC // 2, pair, 0)
    pltpu.sync_copy(acc_v.at[pl.ds(0, N)], o_hbm.at[:, pl.ds(col, 16)])


def _sc_scatter(m, t, dst):
    mesh = plsc.VectorSubcoreMesh(core_axis_name="c", subcore_axis_name="s")
    kern = pl.kernel(
        _sc_scatter_kernel,
        out_type=jax.ShapeDtypeStruct((N, D), _f32),
        mesh=mesh,
        scratch_types=[
            pltpu.VMEM((OUT_ROWS, 16), _f32),
            pltpu.VMEM((SCH, 16), _f32),
            pltpu.VMEM((SCH, 16), _f32),
            pltpu.VMEM((E_PAD,), _i32),
            pltpu.SemaphoreType.DMA,
            pltpu.SemaphoreType.DMA,
        ],
        compiler_params=pltpu.CompilerParams(
            needs_layout_passes=False, use_tc_tiling_on_sc=False),
    )
    return kern(m, t, dst)


def kernel(_input, dependency_triples, W_self, b_self, W_dep, b_dep, W_ff, b_ff):
    src, dst, wid = _plan_edges(dependency_triples)
    t1 = _dense_mm(_input, W_self[0], b_self[0], False)
    g1 = _sc_gather(_input, src)
    m1 = _grouped_mm(g1, W_dep[0], b_dep[0], wid, False)
    x1, g2 = _sc_scatter_gather(m1, t1, dst, src)
    t2 = _dense_mm(x1, W_self[1], b_self[1], True)
    m2 = _grouped_mm(g2, W_dep[1], b_dep[1], wid, True)
    x2 = _sc_scatter(m2, t2, dst)
    return _dense_mm(x2, W_ff, b_ff, True)
